# Initial kernel scaffold; baseline (speedup 1.0000x reference)
import jax
import jax.numpy as jnp
from jax.experimental import pallas as pl


def kernel(session_items, session_len, session_adj, global_adj, emb, Wg, bg, w_ih, w_hh, b_ih, b_hh, W_ein, b_ein, W_eout, b_eout, pos_emb, in_proj_w, in_proj_b, out_proj_w, out_proj_b, W_target, W_3):
    raise NotImplementedError("write your pallas kernel here")


# speedup vs baseline: 1.6322x; 1.6322x over previous
import functools

import jax
import jax.numpy as jnp
import numpy as np
from jax import lax
from jax.experimental import pallas as pl
from jax.experimental.pallas import tpu as pltpu
from jax.experimental.pallas import tpu_sc as plsc

B, L, N, H, NH = 16, 50, 10000, 128, 4
HD = H // NH
BL = B * L
NW = 32
BLP = 1024
RPW = BLP // NW
CH = 8
NCH = RPW // CH
MBLK = 512
MP = 10240
NEG = -1e30

_sc_mesh = plsc.VectorSubcoreMesh(core_axis_name="c", subcore_axis_name="s")


@functools.partial(
    pl.kernel,
    out_type=(
        jax.ShapeDtypeStruct((BLP, H), jnp.float32),
        jax.ShapeDtypeStruct((BLP, H), jnp.float32),
    ),
    mesh=_sc_mesh,
    scratch_types=[
        pltpu.VMEM((16,), jnp.int32),
        pltpu.VMEM((16,), jnp.int32),
        pltpu.VMEM((16, H), jnp.float32),
        pltpu.VMEM((16, H), jnp.float32),
        pltpu.SemaphoreType.DMA,
    ],
)
def _sc_gather(emb_hbm, pos_hbm, idx_hbm, rev_hbm,
               emb_out, pos_out, idx_v, rev_v, erow_v, prow_v, sem):
    wid = lax.axis_index("s") * 2 + lax.axis_index("c")
    base = wid * RPW
    for half in range(RPW // 16):
        b = base + half * 16
        pltpu.sync_copy(idx_hbm.at[pl.ds(b, 16)], idx_v)
        pltpu.sync_copy(rev_hbm.at[pl.ds(b, 16)], rev_v)
        pltpu.async_copy(emb_hbm.at[idx_v], erow_v, sem).wait()
        pltpu.sync_copy(erow_v, emb_out.at[pl.ds(b, 16)])
        pltpu.async_copy(pos_hbm.at[rev_v], prow_v, sem).wait()
        pltpu.sync_copy(prow_v, pos_out.at[pl.ds(b, 16)])


def _dgT(x, w):
    return lax.dot_general(x, w, (((1,), (1,)), ((), ())),
                           preferred_element_type=jnp.float32)


def _dg(x, w):
    return lax.dot_general(x, w, (((1,), (0,)), ((), ())),
                           preferred_element_type=jnp.float32)


GRB = 128


def _spmm_body(idx_ref, adj_any, emb_ref, wg_ref, bg_ref, out_ref,
               rows_v, sem):
    i = pl.program_id(0)
    for r in range(GRB):
        pltpu.make_async_copy(
            adj_any.at[pl.ds(idx_ref[i * GRB + r], 1), :],
            rows_v.at[pl.ds(r, 1), :], sem).start()
    for r in range(GRB):
        pltpu.make_async_copy(
            adj_any.at[pl.ds(0, 1), :],
            rows_v.at[pl.ds(r, 1), :], sem).wait()
    g = jnp.dot(rows_v[...], emb_ref[...], preferred_element_type=jnp.float32)
    out_ref[...] = jnp.maximum(_dgT(g, wg_ref[...]) + bg_ref[...], 0.0)


def _sess_body(sgl_ref, hid_ref, pos_ref, sadj_ref, items_ref, lidx_ref,
               wein_ref, bein_ref, weout_ref, beout_ref, wih_ref, whh_ref,
               bih_ref, bhh_ref, wq_ref, wk_ref, wv_ref, bq_ref, bk_ref,
               bv_ref, wo_ref, bo_ref, wt_ref, w3b_ref, w3c_ref,
               fin_ref, finw_ref, rv_ref):
    bidx = pl.program_id(0)
    hid = hid_ref[0]
    A = sadj_ref[0]
    ein = _dgT(hid, wein_ref[...]) + bein_ref[...]
    eout = _dgT(hid, weout_ref[...]) + beout_ref[...]
    input_in = _dg(A, ein)
    input_out = _dg(A, eout)
    inputs = jnp.concatenate([input_in, input_out], axis=1)
    gi = _dgT(inputs, wih_ref[...]) + bih_ref[...]
    gh = _dgT(hid, whh_ref[...]) + bhh_ref[...]
    i_r, i_i, i_n = gi[:, :H], gi[:, H:2 * H], gi[:, 2 * H:]
    h_r, h_i, h_n = gh[:, :H], gh[:, H:2 * H], gh[:, 2 * H:]
    rg = jax.nn.sigmoid(i_r + h_r)
    ig = jax.nn.sigmoid(i_i + h_i)
    ng = jnp.tanh(i_n + rg * h_n)
    hy = ng + ig * (hid - ng)
    final = sgl_ref[0] + hy + pos_ref[0]
    fin_ref[0] = final
    finw_ref[0] = _dg(final, wt_ref[...])
    li = lidx_ref[bidx]
    rows = lax.broadcasted_iota(jnp.int32, (L, 1), 0)
    sel = (rows == li).astype(jnp.float32)
    last = jnp.sum(final * sel, axis=0, keepdims=True)
    kpm = items_ref[0] == 0
    q = _dgT(last, wq_ref[...]) + bq_ref[...]
    kk = _dgT(final, wk_ref[...]) + bk_ref[...]
    vv = _dgT(final, wv_ref[...]) + bv_ref[...]
    scale = float(1.0 / np.sqrt(HD))
    aos = []
    for h in range(NH):
        qh = q[:, h * HD:(h + 1) * HD]
        kh = kk[:, h * HD:(h + 1) * HD]
        vh = vv[:, h * HD:(h + 1) * HD]
        lg = _dgT(qh, kh) * scale
        lg = jnp.where(kpm, NEG, lg)
        m = jnp.max(lg, axis=1, keepdims=True)
        e = jnp.exp(lg - m)
        a = e / jnp.sum(e, axis=1, keepdims=True)
        aos.append(_dg(a, vh))
    ao = jnp.concatenate(aos, axis=1)
    s_global = _dgT(ao, wo_ref[...]) + bo_ref[...]
    rv_ref[0] = _dgT(last, w3b_ref[...]) + _dgT(s_global, w3c_ref[...])


def _score_body(cand_ref, fin_ref, finw_ref, rv_ref, items_ref, w3a_ref,
                out_ref):
    cand = cand_ref[...]
    candp = _dg(cand, w3a_ref[...])
    base = _dgT(rv_ref[...], cand)
    rows = []
    for b in range(B):
        mask = items_ref[b] == 0
        ts = _dgT(finw_ref[b], cand)
        ts = jnp.where(mask, NEG, ts)
        m = jnp.max(ts, axis=0, keepdims=True)
        e = jnp.exp(ts - m)
        a = e / jnp.sum(e, axis=0, keepdims=True)
        fp = _dgT(fin_ref[b], candp)
        rows.append(jnp.sum(a * fp, axis=0, keepdims=True) + base[b:b + 1, :])
    out_ref[...] = jnp.concatenate(rows, axis=0)


def kernel(session_items, session_len, session_adj, global_adj, emb, Wg, bg,
           w_ih, w_hh, b_ih, b_hh, W_ein, b_ein, W_eout, b_eout, pos_emb,
           in_proj_w, in_proj_b, out_proj_w, out_proj_b, W_target, W_3):
    session_items = session_items.astype(jnp.int32)
    session_len = session_len.astype(jnp.int32)
    f32 = jnp.float32

    idx = session_items.reshape(-1)
    idxp = jnp.concatenate([idx, jnp.zeros((BLP - BL,), jnp.int32)])
    positions = jnp.arange(L, dtype=jnp.int32)[None, :]
    rev = session_len[:, None] - 1 - positions
    rev = jnp.where(session_items == 0, 0, rev)
    rev = jnp.clip(rev, 0, 199).reshape(-1)
    revp = jnp.concatenate([rev, jnp.zeros((BLP - BL,), jnp.int32)])

    hid_flat, pos_flat = _sc_gather(
        emb.astype(f32), pos_emb.astype(f32), idxp, revp)

    sgf = pl.pallas_call(
        _spmm_body,
        grid_spec=pltpu.PrefetchScalarGridSpec(
            num_scalar_prefetch=1,
            grid=(BLP // GRB,),
            in_specs=[
                pl.BlockSpec(memory_space=pltpu.MemorySpace.HBM),
                pl.BlockSpec((N, H), lambda i, idx: (0, 0)),
                pl.BlockSpec((H, H), lambda i, idx: (0, 0)),
                pl.BlockSpec((1, H), lambda i, idx: (0, 0)),
            ],
            out_specs=pl.BlockSpec((GRB, H), lambda i, idx: (i, 0)),
            scratch_shapes=[
                pltpu.VMEM((GRB, N), f32),
                pltpu.SemaphoreType.DMA,
            ],
        ),
        out_shape=jax.ShapeDtypeStruct((BLP, H), f32),
    )(idxp, global_adj, emb, Wg, bg.reshape(1, H))

    sgl = sgf[:BL].reshape(B, L, H)
    hid = hid_flat[:BL].reshape(B, L, H)
    pos = pos_flat[:BL].reshape(B, L, H)
    items_row = session_items.reshape(B, 1, L)
    items_col = session_items.reshape(B, L, 1)
    last_idx = jnp.clip(session_len - 1, 0, L - 1)

    Wq, Wk, Wv = in_proj_w[:H], in_proj_w[H:2 * H], in_proj_w[2 * H:]
    bq = in_proj_b[:H].reshape(1, H)
    bk = in_proj_b[H:2 * H].reshape(1, H)
    bv = in_proj_b[2 * H:].reshape(1, H)
    W3a, W3b, W3c = W_3[:, :H], W_3[:, H:2 * H], W_3[:, 2 * H:]

    full = lambda shp: pl.BlockSpec(shp, lambda i: tuple(0 for _ in shp))
    fin, finw, rv3 = pl.pallas_call(
        _sess_body,
        grid=(B,),
        in_specs=[
            pl.BlockSpec((1, L, H), lambda i: (i, 0, 0)),
            pl.BlockSpec((1, L, H), lambda i: (i, 0, 0)),
            pl.BlockSpec((1, L, H), lambda i: (i, 0, 0)),
            pl.BlockSpec((1, L, L), lambda i: (i, 0, 0)),
            pl.BlockSpec((1, 1, L), lambda i: (i, 0, 0)),
            pl.BlockSpec(memory_space=pltpu.SMEM),
            full((H, H)), full((1, H)), full((H, H)), full((1, H)),
            full((3 * H, 2 * H)), full((3 * H, H)),
            full((1, 3 * H)), full((1, 3 * H)),
            full((H, H)), full((H, H)), full((H, H)),
            full((1, H)), full((1, H)), full((1, H)),
            full((H, H)), full((1, H)), full((H, H)),
            full((H, H)), full((H, H)),
        ],
        out_specs=[
            pl.BlockSpec((1, L, H), lambda i: (i, 0, 0)),
            pl.BlockSpec((1, L, H), lambda i: (i, 0, 0)),
            pl.BlockSpec((1, 1, H), lambda i: (i, 0, 0)),
        ],
        out_shape=[
            jax.ShapeDtypeStruct((B, L, H), f32),
            jax.ShapeDtypeStruct((B, L, H), f32),
            jax.ShapeDtypeStruct((B, 1, H), f32),
        ],
    )(sgl, hid, pos, session_adj, items_row, last_idx,
      W_ein, b_ein.reshape(1, H), W_eout, b_eout.reshape(1, H),
      w_ih, w_hh, b_ih.reshape(1, 3 * H), b_hh.reshape(1, 3 * H),
      Wq, Wk, Wv, bq, bk, bv,
      out_proj_w, out_proj_b.reshape(1, H), W_target, W3b, W3c)

    rv = rv3.reshape(B, H)
    cand = jnp.concatenate(
        [emb[1:], jnp.zeros((MP - (N - 1), H), f32)], axis=0)

    scores_p = pl.pallas_call(
        _score_body,
        grid=(MP // MBLK,),
        in_specs=[
            pl.BlockSpec((MBLK, H), lambda i: (i, 0)),
            pl.BlockSpec((B, L, H), lambda i: (0, 0, 0)),
            pl.BlockSpec((B, L, H), lambda i: (0, 0, 0)),
            pl.BlockSpec((B, H), lambda i: (0, 0)),
            pl.BlockSpec((B, L, 1), lambda i: (0, 0, 0)),
            pl.BlockSpec((H, H), lambda i: (0, 0)),
        ],
        out_specs=pl.BlockSpec((B, MBLK), lambda i: (0, i)),
        out_shape=jax.ShapeDtypeStruct((B, MP), f32),
    )(cand, fin, finw, rv, items_col, W3a)

    return scores_p[:, :N - 1]


# speedup vs baseline: 2.2012x; 1.3486x over previous
import functools

import jax
import jax.numpy as jnp
import numpy as np
from jax import lax
from jax.experimental import pallas as pl
from jax.experimental.pallas import tpu as pltpu
from jax.experimental.pallas import tpu_sc as plsc

B, L, N, H, NH = 16, 50, 10000, 128, 4
HD = H // NH
LP = 64
BLP = B * LP
NW = 32
RPW = BLP // NW
GRB = 128
MBLK = 512
MP = 10240
NEG = -1e30

_sc_mesh = plsc.VectorSubcoreMesh(core_axis_name="c", subcore_axis_name="s")


@functools.partial(
    pl.kernel,
    out_type=(
        jax.ShapeDtypeStruct((BLP, H), jnp.float32),
        jax.ShapeDtypeStruct((BLP, H), jnp.float32),
    ),
    mesh=_sc_mesh,
    scratch_types=[
        pltpu.VMEM((RPW,), jnp.int32),
        pltpu.VMEM((RPW,), jnp.int32),
        pltpu.VMEM((RPW, H), jnp.float32),
        pltpu.VMEM((RPW, H), jnp.float32),
        pltpu.SemaphoreType.DMA,
        pltpu.SemaphoreType.DMA,
    ],
)
def _sc_gather(emb_hbm, pos_hbm, idx_hbm, rev_hbm,
               emb_out, pos_out, idx_v, rev_v, erow_v, prow_v, sem, sem2):
    wid = lax.axis_index("s") * 2 + lax.axis_index("c")
    b = wid * RPW
    pltpu.sync_copy(idx_hbm.at[pl.ds(b, RPW)], idx_v)
    pltpu.sync_copy(rev_hbm.at[pl.ds(b, RPW)], rev_v)
    ce = pltpu.async_copy(emb_hbm.at[idx_v], erow_v, sem)
    cp = pltpu.async_copy(pos_hbm.at[rev_v], prow_v, sem2)
    ce.wait()
    cp.wait()
    pltpu.sync_copy(erow_v, emb_out.at[pl.ds(b, RPW)])
    pltpu.sync_copy(prow_v, pos_out.at[pl.ds(b, RPW)])


def _dgT(x, w):
    return lax.dot_general(x, w, (((1,), (1,)), ((), ())),
                           preferred_element_type=jnp.float32)


def _dg(x, w):
    return lax.dot_general(x, w, (((1,), (0,)), ((), ())),
                           preferred_element_type=jnp.float32)


def _spmm_body(idx_ref, adj_any, emb_ref, wg_ref, bg_ref, out_ref,
               rows_v, sems):
    i = pl.program_id(0)
    nb = pl.num_programs(0)

    def issue(block, slot):
        for r in range(GRB):
            pltpu.make_async_copy(
                adj_any.at[pl.ds(idx_ref[block * GRB + r], 1), :],
                rows_v.at[slot, pl.ds(r, 1), :], sems.at[slot]).start()

    def drain_compute(slot):
        for r in range(GRB):
            pltpu.make_async_copy(
                adj_any.at[pl.ds(0, 1), :],
                rows_v.at[slot, pl.ds(r, 1), :], sems.at[slot]).wait()
        g = jnp.dot(rows_v[slot], emb_ref[...],
                    preferred_element_type=jnp.float32)
        out_ref[...] = jnp.maximum(_dgT(g, wg_ref[...]) + bg_ref[...], 0.0)

    @pl.when(i == 0)
    def _():
        issue(0, 0)

    p = lax.rem(i, 2)

    @pl.when(p == 0)
    def _():
        @pl.when(i + 1 < nb)
        def _():
            issue(i + 1, 1)
        drain_compute(0)

    @pl.when(p == 1)
    def _():
        @pl.when(i + 1 < nb)
        def _():
            issue(i + 1, 0)
        drain_compute(1)


def _sess_body(sgl_ref, hid_ref, pos_ref, sadj_ref, items_ref, lidx_ref,
               wein_ref, bein_ref, weout_ref, beout_ref, wih_ref, whh_ref,
               bih_ref, bhh_ref, wq_ref, wk_ref, wv_ref, bq_ref, bk_ref,
               bv_ref, wo_ref, bo_ref, wt_ref, w3b_ref, w3c_ref,
               fin_ref, finw_ref, rv_ref):
    hid = hid_ref[...]
    ein = _dgT(hid, wein_ref[...]) + bein_ref[...]
    eout = _dgT(hid, weout_ref[...]) + beout_ref[...]
    iis, ios = [], []
    for b in range(B):
        A = sadj_ref[b]
        iis.append(_dg(A, ein[b * LP:(b + 1) * LP]))
        ios.append(_dg(A, eout[b * LP:(b + 1) * LP]))
    inputs = jnp.concatenate(
        [jnp.concatenate(iis, axis=0), jnp.concatenate(ios, axis=0)], axis=1)
    gi = _dgT(inputs, wih_ref[...]) + bih_ref[...]
    gh = _dgT(hid, whh_ref[...]) + bhh_ref[...]
    i_r, i_i, i_n = gi[:, :H], gi[:, H:2 * H], gi[:, 2 * H:]
    h_r, h_i, h_n = gh[:, :H], gh[:, H:2 * H], gh[:, 2 * H:]
    rg = jax.nn.sigmoid(i_r + h_r)
    ig = jax.nn.sigmoid(i_i + h_i)
    ng = jnp.tanh(i_n + rg * h_n)
    hy = ng + ig * (hid - ng)
    final = sgl_ref[...] + hy + pos_ref[...]
    fin_ref[...] = final
    finw_ref[...] = _dg(final, wt_ref[...])
    rows = lax.broadcasted_iota(jnp.int32, (LP, 1), 0)
    lasts = []
    for b in range(B):
        li = lidx_ref[b]
        fb = final[b * LP:(b + 1) * LP]
        sel = (rows == li).astype(jnp.float32)
        lasts.append(jnp.sum(fb * sel, axis=0, keepdims=True))
    last = jnp.concatenate(lasts, axis=0)
    q = _dgT(last, wq_ref[...]) + bq_ref[...]
    kk = _dgT(final, wk_ref[...]) + bk_ref[...]
    vv = _dgT(final, wv_ref[...]) + bv_ref[...]
    scale = float(1.0 / np.sqrt(HD))
    aos = []
    for b in range(B):
        kpm = items_ref[b] == 0
        parts = []
        for h in range(NH):
            qh = q[b:b + 1, h * HD:(h + 1) * HD]
            kh = kk[b * LP:(b + 1) * LP, h * HD:(h + 1) * HD]
            vh = vv[b * LP:(b + 1) * LP, h * HD:(h + 1) * HD]
            lg = _dgT(qh, kh) * scale
            lg = jnp.where(kpm, NEG, lg)
            m = jnp.max(lg, axis=1, keepdims=True)
            e = jnp.exp(lg - m)
            a = e / jnp.sum(e, axis=1, keepdims=True)
            parts.append(_dg(a, vh))
        aos.append(jnp.concatenate(parts, axis=1))
    ao = jnp.concatenate(aos, axis=0)
    s_global = _dgT(ao, wo_ref[...]) + bo_ref[...]
    rv_ref[...] = _dgT(last, w3b_ref[...]) + _dgT(s_global, w3c_ref[...])


def _score_body(cand_ref, fin_ref, finw_ref, rv_ref, items_ref, w3a_ref,
                out_ref):
    cand = cand_ref[...]
    candp = _dg(cand, w3a_ref[...])
    base = _dgT(rv_ref[...], cand)
    rows = []
    for b in range(B):
        mask = items_ref[b] == 0
        ts = _dgT(finw_ref[pl.ds(b * LP, LP)], cand)
        ts = jnp.where(mask, NEG, ts)
        m = jnp.max(ts, axis=0, keepdims=True)
        e = jnp.exp(ts - m)
        a = e / jnp.sum(e, axis=0, keepdims=True)
        fp = _dgT(fin_ref[pl.ds(b * LP, LP)], candp)
        rows.append(jnp.sum(a * fp, axis=0, keepdims=True) + base[b:b + 1, :])
    out_ref[...] = jnp.concatenate(rows, axis=0)


def kernel(session_items, session_len, session_adj, global_adj, emb, Wg, bg,
           w_ih, w_hh, b_ih, b_hh, W_ein, b_ein, W_eout, b_eout, pos_emb,
           in_proj_w, in_proj_b, out_proj_w, out_proj_b, W_target, W_3):
    session_items = session_items.astype(jnp.int32)
    session_len = session_len.astype(jnp.int32)
    f32 = jnp.float32

    items_p = jnp.pad(session_items, ((0, 0), (0, LP - L)))
    idxp = items_p.reshape(-1)
    positions = jnp.arange(L, dtype=jnp.int32)[None, :]
    rev = session_len[:, None] - 1 - positions
    rev = jnp.where(session_items == 0, 0, rev)
    rev = jnp.clip(rev, 0, 199)
    revp = jnp.pad(rev, ((0, 0), (0, LP - L))).reshape(-1)
    sadj_p = jnp.pad(session_adj, ((0, 0), (0, LP - L), (0, LP - L)))
    items3 = items_p.reshape(B, 1, LP)
    items_col = items_p.reshape(B, LP, 1)
    last_idx = jnp.clip(session_len - 1, 0, L - 1)

    hid_flat, pos_flat = _sc_gather(
        emb.astype(f32), pos_emb.astype(f32), idxp, revp)

    sgf = pl.pallas_call(
        _spmm_body,
        grid_spec=pltpu.PrefetchScalarGridSpec(
            num_scalar_prefetch=1,
            grid=(BLP // GRB,),
            in_specs=[
                pl.BlockSpec(memory_space=pltpu.MemorySpace.HBM),
                pl.BlockSpec((N, H), lambda i, idx: (0, 0)),
                pl.BlockSpec((H, H), lambda i, idx: (0, 0)),
                pl.BlockSpec((1, H), lambda i, idx: (0, 0)),
            ],
            out_specs=pl.BlockSpec((GRB, H), lambda i, idx: (i, 0)),
            scratch_shapes=[
                pltpu.VMEM((2, GRB, N), f32),
                pltpu.SemaphoreType.DMA((2,)),
            ],
        ),
        out_shape=jax.ShapeDtypeStruct((BLP, H), f32),
    )(idxp, global_adj, emb, Wg, bg.reshape(1, H))

    Wq, Wk, Wv = in_proj_w[:H], in_proj_w[H:2 * H], in_proj_w[2 * H:]
    bq = in_proj_b[:H].reshape(1, H)
    bk = in_proj_b[H:2 * H].reshape(1, H)
    bv = in_proj_b[2 * H:].reshape(1, H)
    W3a, W3b, W3c = W_3[:, :H], W_3[:, H:2 * H], W_3[:, 2 * H:]

    full = lambda shp: pl.BlockSpec(shp, lambda: tuple(0 for _ in shp))
    fin, finw, rv = pl.pallas_call(
        _sess_body,
        in_specs=[
            full((BLP, H)), full((BLP, H)), full((BLP, H)),
            full((B, LP, LP)), full((B, 1, LP)),
            pl.BlockSpec(memory_space=pltpu.SMEM),
            full((H, H)), full((1, H)), full((H, H)), full((1, H)),
            full((3 * H, 2 * H)), full((3 * H, H)),
            full((1, 3 * H)), full((1, 3 * H)),
            full((H, H)), full((H, H)), full((H, H)),
            full((1, H)), full((1, H)), full((1, H)),
            full((H, H)), full((1, H)), full((H, H)),
            full((H, H)), full((H, H)),
        ],
        out_specs=[
            full((BLP, H)), full((BLP, H)), full((B, H)),
        ],
        out_shape=[
            jax.ShapeDtypeStruct((BLP, H), f32),
            jax.ShapeDtypeStruct((BLP, H), f32),
            jax.ShapeDtypeStruct((B, H), f32),
        ],
    )(sgf, hid_flat, pos_flat, sadj_p, items3, last_idx,
      W_ein, b_ein.reshape(1, H), W_eout, b_eout.reshape(1, H),
      w_ih, w_hh, b_ih.reshape(1, 3 * H), b_hh.reshape(1, 3 * H),
      Wq, Wk, Wv, bq, bk, bv,
      out_proj_w, out_proj_b.reshape(1, H), W_target, W3b, W3c)

    cand = jnp.concatenate(
        [emb[1:], jnp.zeros((MP - (N - 1), H), f32)], axis=0)

    scores_p = pl.pallas_call(
        _score_body,
        grid=(MP // MBLK,),
        in_specs=[
            pl.BlockSpec((MBLK, H), lambda i: (i, 0)),
            pl.BlockSpec((BLP, H), lambda i: (0, 0)),
            pl.BlockSpec((BLP, H), lambda i: (0, 0)),
            pl.BlockSpec((B, H), lambda i: (0, 0)),
            pl.BlockSpec((B, LP, 1), lambda i: (0, 0, 0)),
            pl.BlockSpec((H, H), lambda i: (0, 0)),
        ],
        out_specs=pl.BlockSpec((B, MBLK), lambda i: (0, i)),
        out_shape=jax.ShapeDtypeStruct((B, MP), f32),
    )(cand, fin, finw, rv, items_col, W3a)

    return scores_p[:, :N - 1]


# speedup vs baseline: 2.7867x; 1.2660x over previous
import functools

import jax
import jax.numpy as jnp
import numpy as np
from jax import lax
from jax.experimental import pallas as pl
from jax.experimental.pallas import tpu as pltpu
from jax.experimental.pallas import tpu_sc as plsc

B, L, N, H, NH = 16, 50, 10000, 128, 4
HD = H // NH
LP = 64
BLP = B * LP
NW = 32
RPW = BLP // NW
GRB = 128
MBLK = 1024
MP = 10240
NEG = -1e30

_sc_mesh = plsc.VectorSubcoreMesh(core_axis_name="c", subcore_axis_name="s")


@functools.partial(
    pl.kernel,
    out_type=(
        jax.ShapeDtypeStruct((BLP, H), jnp.float32),
        jax.ShapeDtypeStruct((BLP, H), jnp.float32),
    ),
    mesh=_sc_mesh,
    scratch_types=[
        pltpu.VMEM((RPW,), jnp.int32),
        pltpu.VMEM((RPW,), jnp.int32),
        pltpu.VMEM((RPW, H), jnp.float32),
        pltpu.VMEM((RPW, H), jnp.float32),
        pltpu.SemaphoreType.DMA,
        pltpu.SemaphoreType.DMA,
    ],
)
def _sc_gather(emb_hbm, pos_hbm, idx_hbm, rev_hbm,
               emb_out, pos_out, idx_v, rev_v, erow_v, prow_v, sem, sem2):
    wid = lax.axis_index("s") * 2 + lax.axis_index("c")
    b = wid * RPW
    pltpu.sync_copy(idx_hbm.at[pl.ds(b, RPW)], idx_v)
    pltpu.sync_copy(rev_hbm.at[pl.ds(b, RPW)], rev_v)
    ce = pltpu.async_copy(emb_hbm.at[idx_v], erow_v, sem)
    cp = pltpu.async_copy(pos_hbm.at[rev_v], prow_v, sem2)
    ce.wait()
    cp.wait()
    pltpu.sync_copy(erow_v, emb_out.at[pl.ds(b, RPW)])
    pltpu.sync_copy(prow_v, pos_out.at[pl.ds(b, RPW)])


def _dgT(x, w):
    return lax.dot_general(x, w, (((1,), (1,)), ((), ())),
                           preferred_element_type=jnp.float32)


def _dg(x, w):
    return lax.dot_general(x, w, (((1,), (0,)), ((), ())),
                           preferred_element_type=jnp.float32)


def _spmm_body(idx_ref, adj_any, emb_ref, wg_ref, bg_ref, out_ref,
               rows_v, sems):
    i = pl.program_id(0)
    nb = pl.num_programs(0)

    def issue(block, slot):
        for r in range(GRB):
            if r % LP < L:
                pltpu.make_async_copy(
                    adj_any.at[pl.ds(idx_ref[block * GRB + r], 1), :],
                    rows_v.at[slot, pl.ds(r, 1), :], sems.at[slot]).start()

    def drain_compute(slot):
        for r in range(GRB):
            if r % LP < L:
                pltpu.make_async_copy(
                    adj_any.at[pl.ds(0, 1), :],
                    rows_v.at[slot, pl.ds(r, 1), :], sems.at[slot]).wait()
        g = jnp.dot(rows_v[slot], emb_ref[...],
                    preferred_element_type=jnp.float32)
        val = jnp.maximum(_dgT(g, wg_ref[...]) + bg_ref[...], 0.0)
        rid = lax.broadcasted_iota(jnp.int32, (GRB, 1), 0)
        out_ref[...] = jnp.where(rid % LP < L, val, 0.0)

    @pl.when(i == 0)
    def _():
        issue(0, 0)

    p = lax.rem(i, 2)

    @pl.when(p == 0)
    def _():
        @pl.when(i + 1 < nb)
        def _():
            issue(i + 1, 1)
        drain_compute(0)

    @pl.when(p == 1)
    def _():
        @pl.when(i + 1 < nb)
        def _():
            issue(i + 1, 0)
        drain_compute(1)


def _sess_body(sgl_ref, hid_ref, pos_ref, sadj_ref, items_ref, lidx_ref,
               wein_ref, bein_ref, weout_ref, beout_ref, wih_ref, whh_ref,
               bih_ref, bhh_ref, wq_ref, wk_ref, wv_ref, bq_ref, bk_ref,
               bv_ref, wo_ref, bo_ref, wt_ref, w3b_ref, w3c_ref,
               fin_ref, finw_ref, rv_ref):
    hid = hid_ref[...]
    ein = _dgT(hid, wein_ref[...]) + bein_ref[...]
    eout = _dgT(hid, weout_ref[...]) + beout_ref[...]
    iis, ios = [], []
    for b in range(B):
        A = sadj_ref[b]
        iis.append(_dg(A, ein[b * LP:(b + 1) * LP]))
        ios.append(_dg(A, eout[b * LP:(b + 1) * LP]))
    inputs = jnp.concatenate(
        [jnp.concatenate(iis, axis=0), jnp.concatenate(ios, axis=0)], axis=1)
    gi = _dgT(inputs, wih_ref[...]) + bih_ref[...]
    gh = _dgT(hid, whh_ref[...]) + bhh_ref[...]
    i_r, i_i, i_n = gi[:, :H], gi[:, H:2 * H], gi[:, 2 * H:]
    h_r, h_i, h_n = gh[:, :H], gh[:, H:2 * H], gh[:, 2 * H:]
    rg = jax.nn.sigmoid(i_r + h_r)
    ig = jax.nn.sigmoid(i_i + h_i)
    ng = jnp.tanh(i_n + rg * h_n)
    hy = ng + ig * (hid - ng)
    final = sgl_ref[...] + hy + pos_ref[...]
    fin_ref[...] = final
    finw_ref[...] = _dg(final, wt_ref[...])
    rowid = lax.broadcasted_iota(jnp.int32, (B, BLP), 0)
    col = lax.broadcasted_iota(jnp.int32, (B, BLP), 1)
    valid = (col // LP == rowid) & (items_ref[...] != 0)
    targets = lidx_ref[...] + LP * lax.broadcasted_iota(jnp.int32, (B, 1), 0)
    onehot = (col == targets).astype(jnp.float32)
    last = _dg(onehot, final)
    q = _dgT(last, wq_ref[...]) + bq_ref[...]
    kk = _dgT(final, wk_ref[...]) + bk_ref[...]
    vv = _dgT(final, wv_ref[...]) + bv_ref[...]
    scale = float(1.0 / np.sqrt(HD))
    parts = []
    for h in range(NH):
        hs = slice(h * HD, (h + 1) * HD)
        lg = _dgT(q[:, hs], kk[:, hs]) * scale
        lg = jnp.where(valid, lg, NEG)
        m = jnp.max(lg, axis=1, keepdims=True)
        e = jnp.exp(lg - m)
        a = e / jnp.sum(e, axis=1, keepdims=True)
        parts.append(_dg(a, vv[:, hs]))
    ao = jnp.concatenate(parts, axis=1)
    s_global = _dgT(ao, wo_ref[...]) + bo_ref[...]
    rv_ref[...] = _dgT(last, w3b_ref[...]) + _dgT(s_global, w3c_ref[...])


def _score_body(cand_ref, fin_ref, finw_ref, rv_ref, items_ref, w3a_ref,
                out_ref):
    cand = cand_ref[...]
    candp = _dg(cand, w3a_ref[...])
    base = _dgT(rv_ref[...], cand)
    rows = []
    for b in range(B):
        mask = items_ref[b] == 0
        ts = _dgT(finw_ref[pl.ds(b * LP, LP)], cand)
        ts = jnp.where(mask, NEG, ts)
        m = jnp.max(ts, axis=0, keepdims=True)
        e = jnp.exp(ts - m)
        a = e / jnp.sum(e, axis=0, keepdims=True)
        fp = _dgT(fin_ref[pl.ds(b * LP, LP)], candp)
        rows.append(jnp.sum(a * fp, axis=0, keepdims=True) + base[b:b + 1, :])
    out_ref[...] = jnp.concatenate(rows, axis=0)


def kernel(session_items, session_len, session_adj, global_adj, emb, Wg, bg,
           w_ih, w_hh, b_ih, b_hh, W_ein, b_ein, W_eout, b_eout, pos_emb,
           in_proj_w, in_proj_b, out_proj_w, out_proj_b, W_target, W_3):
    session_items = session_items.astype(jnp.int32)
    session_len = session_len.astype(jnp.int32)
    f32 = jnp.float32

    items_p = jnp.pad(session_items, ((0, 0), (0, LP - L)))
    idxp = items_p.reshape(-1)
    positions = jnp.arange(L, dtype=jnp.int32)[None, :]
    rev = session_len[:, None] - 1 - positions
    rev = jnp.where(session_items == 0, 0, rev)
    rev = jnp.clip(rev, 0, 199)
    revp = jnp.pad(rev, ((0, 0), (0, LP - L))).reshape(-1)
    sadj_p = jnp.pad(session_adj, ((0, 0), (0, LP - L), (0, LP - L)))
    items_col = items_p.reshape(B, LP, 1)
    last_idx = jnp.clip(session_len - 1, 0, L - 1)

    hid_flat, pos_flat = _sc_gather(
        emb.astype(f32), pos_emb.astype(f32), idxp, revp)

    sgf = pl.pallas_call(
        _spmm_body,
        grid_spec=pltpu.PrefetchScalarGridSpec(
            num_scalar_prefetch=1,
            grid=(BLP // GRB,),
            in_specs=[
                pl.BlockSpec(memory_space=pltpu.MemorySpace.HBM),
                pl.BlockSpec((N, H), lambda i, idx: (0, 0)),
                pl.BlockSpec((H, H), lambda i, idx: (0, 0)),
                pl.BlockSpec((1, H), lambda i, idx: (0, 0)),
            ],
            out_specs=pl.BlockSpec((GRB, H), lambda i, idx: (i, 0)),
            scratch_shapes=[
                pltpu.VMEM((2, GRB, N), f32),
                pltpu.SemaphoreType.DMA((2,)),
            ],
        ),
        out_shape=jax.ShapeDtypeStruct((BLP, H), f32),
    )(idxp, global_adj, emb, Wg, bg.reshape(1, H))

    Wq, Wk, Wv = in_proj_w[:H], in_proj_w[H:2 * H], in_proj_w[2 * H:]
    bq = in_proj_b[:H].reshape(1, H)
    bk = in_proj_b[H:2 * H].reshape(1, H)
    bv = in_proj_b[2 * H:].reshape(1, H)
    W3a, W3b, W3c = W_3[:, :H], W_3[:, H:2 * H], W_3[:, 2 * H:]

    full = lambda shp: pl.BlockSpec(shp, lambda: tuple(0 for _ in shp))
    fin, finw, rv = pl.pallas_call(
        _sess_body,
        in_specs=[
            full((BLP, H)), full((BLP, H)), full((BLP, H)),
            full((B, LP, LP)), full((1, BLP)), full((B, 1)),
            full((H, H)), full((1, H)), full((H, H)), full((1, H)),
            full((3 * H, 2 * H)), full((3 * H, H)),
            full((1, 3 * H)), full((1, 3 * H)),
            full((H, H)), full((H, H)), full((H, H)),
            full((1, H)), full((1, H)), full((1, H)),
            full((H, H)), full((1, H)), full((H, H)),
            full((H, H)), full((H, H)),
        ],
        out_specs=[
            full((BLP, H)), full((BLP, H)), full((B, H)),
        ],
        out_shape=[
            jax.ShapeDtypeStruct((BLP, H), f32),
            jax.ShapeDtypeStruct((BLP, H), f32),
            jax.ShapeDtypeStruct((B, H), f32),
        ],
    )(sgf, hid_flat, pos_flat, sadj_p, items_p.reshape(1, BLP),
      last_idx.reshape(B, 1),
      W_ein, b_ein.reshape(1, H), W_eout, b_eout.reshape(1, H),
      w_ih, w_hh, b_ih.reshape(1, 3 * H), b_hh.reshape(1, 3 * H),
      Wq, Wk, Wv, bq, bk, bv,
      out_proj_w, out_proj_b.reshape(1, H), W_target, W3b, W3c)

    cand = jnp.concatenate(
        [emb[1:], jnp.zeros((MP - (N - 1), H), f32)], axis=0)

    scores_p = pl.pallas_call(
        _score_body,
        grid=(MP // MBLK,),
        in_specs=[
            pl.BlockSpec((MBLK, H), lambda i: (i, 0)),
            pl.BlockSpec((BLP, H), lambda i: (0, 0)),
            pl.BlockSpec((BLP, H), lambda i: (0, 0)),
            pl.BlockSpec((B, H), lambda i: (0, 0)),
            pl.BlockSpec((B, LP, 1), lambda i: (0, 0, 0)),
            pl.BlockSpec((H, H), lambda i: (0, 0)),
        ],
        out_specs=pl.BlockSpec((B, MBLK), lambda i: (0, i)),
        out_shape=jax.ShapeDtypeStruct((B, MP), f32),
    )(cand, fin, finw, rv, items_col, W3a)

    return scores_p[:, :N - 1]


# speedup vs baseline: 2.9692x; 1.0655x over previous
import functools

import jax
import jax.numpy as jnp
import numpy as np
from jax import lax
from jax.experimental import pallas as pl
from jax.experimental.pallas import tpu as pltpu
from jax.experimental.pallas import tpu_sc as plsc

B, L, N, H, NH = 16, 50, 10000, 128, 4
HD = H // NH
LP = 64
BLP = B * LP
NW = 32
RPW = BLP // NW
GRB = 128
MBLK = 1024
MP = 10240
NEG = -1e30

_sc_mesh = plsc.VectorSubcoreMesh(core_axis_name="c", subcore_axis_name="s")


@functools.partial(
    pl.kernel,
    out_type=(
        jax.ShapeDtypeStruct((BLP, H), jnp.float32),
        jax.ShapeDtypeStruct((BLP, H), jnp.float32),
    ),
    mesh=_sc_mesh,
    scratch_types=[
        pltpu.VMEM((RPW,), jnp.int32),
        pltpu.VMEM((RPW,), jnp.int32),
        pltpu.VMEM((RPW, H), jnp.float32),
        pltpu.VMEM((RPW, H), jnp.float32),
        pltpu.SemaphoreType.DMA,
        pltpu.SemaphoreType.DMA,
    ],
)
def _sc_gather(emb_hbm, pos_hbm, idx_hbm, rev_hbm,
               emb_out, pos_out, idx_v, rev_v, erow_v, prow_v, sem, sem2):
    wid = lax.axis_index("s") * 2 + lax.axis_index("c")
    b = wid * RPW
    pltpu.sync_copy(idx_hbm.at[pl.ds(b, RPW)], idx_v)
    pltpu.sync_copy(rev_hbm.at[pl.ds(b, RPW)], rev_v)
    ce = pltpu.async_copy(emb_hbm.at[idx_v], erow_v, sem)
    cp = pltpu.async_copy(pos_hbm.at[rev_v], prow_v, sem2)
    ce.wait()
    cp.wait()
    pltpu.sync_copy(erow_v, emb_out.at[pl.ds(b, RPW)])
    pltpu.sync_copy(prow_v, pos_out.at[pl.ds(b, RPW)])


def _dgT(x, w):
    return lax.dot_general(x, w, (((1,), (1,)), ((), ())),
                           preferred_element_type=jnp.float32)


def _dg(x, w):
    return lax.dot_general(x, w, (((1,), (0,)), ((), ())),
                           preferred_element_type=jnp.float32)


def _spmm_body(idx_ref, adj_any, emb_ref, wg_ref, bg_ref, out_ref,
               rows_v, sems):
    i = pl.program_id(0)
    nb = pl.num_programs(0)

    def issue(block, slot):
        for r in range(GRB):
            if r % LP < L:
                pltpu.make_async_copy(
                    adj_any.at[pl.ds(idx_ref[block * GRB + r], 1), :],
                    rows_v.at[slot, pl.ds(r, 1), :], sems.at[slot]).start()

    def drain_compute(slot):
        for r in range(GRB):
            if r % LP < L:
                pltpu.make_async_copy(
                    adj_any.at[pl.ds(0, 1), :],
                    rows_v.at[slot, pl.ds(r, 1), :], sems.at[slot]).wait()
        g = jnp.dot(rows_v[slot], emb_ref[...],
                    preferred_element_type=jnp.float32)
        val = jnp.maximum(_dgT(g, wg_ref[...]) + bg_ref[...], 0.0)
        rid = lax.broadcasted_iota(jnp.int32, (GRB, 1), 0)
        out_ref[...] = jnp.where(rid % LP < L, val, 0.0)

    @pl.when(i == 0)
    def _():
        issue(0, 0)

    p = lax.rem(i, 2)

    @pl.when(p == 0)
    def _():
        @pl.when(i + 1 < nb)
        def _():
            issue(i + 1, 1)
        drain_compute(0)

    @pl.when(p == 1)
    def _():
        @pl.when(i + 1 < nb)
        def _():
            issue(i + 1, 0)
        drain_compute(1)


def _sess_body(sgl_ref, hid_ref, pos_ref, sadj_ref, items_ref, lidx_ref,
               wein_ref, bein_ref, weout_ref, beout_ref, wih_ref, whh_ref,
               bih_ref, bhh_ref, wq_ref, wk_ref, wv_ref, bq_ref, bk_ref,
               bv_ref, wo_ref, bo_ref, wt_ref, w3b_ref, w3c_ref,
               fin_ref, finw_ref, rv_ref):
    hid = hid_ref[...]
    ein = _dgT(hid, wein_ref[...]) + bein_ref[...]
    eout = _dgT(hid, weout_ref[...]) + beout_ref[...]
    iis, ios = [], []
    for b in range(B):
        A = sadj_ref[b]
        iis.append(_dg(A, ein[b * LP:(b + 1) * LP]))
        ios.append(_dg(A, eout[b * LP:(b + 1) * LP]))
    inputs = jnp.concatenate(
        [jnp.concatenate(iis, axis=0), jnp.concatenate(ios, axis=0)], axis=1)
    gi = _dgT(inputs, wih_ref[...]) + bih_ref[...]
    gh = _dgT(hid, whh_ref[...]) + bhh_ref[...]
    i_r, i_i, i_n = gi[:, :H], gi[:, H:2 * H], gi[:, 2 * H:]
    h_r, h_i, h_n = gh[:, :H], gh[:, H:2 * H], gh[:, 2 * H:]
    rg = jax.nn.sigmoid(i_r + h_r)
    ig = jax.nn.sigmoid(i_i + h_i)
    ng = jnp.tanh(i_n + rg * h_n)
    hy = ng + ig * (hid - ng)
    final = sgl_ref[...] + hy + pos_ref[...]
    fin_ref[...] = final
    finw_ref[...] = _dg(final, wt_ref[...])
    rowid = lax.broadcasted_iota(jnp.int32, (B, BLP), 0)
    col = lax.broadcasted_iota(jnp.int32, (B, BLP), 1)
    valid = (col // LP == rowid) & (items_ref[...] != 0)
    targets = lidx_ref[...] + LP * lax.broadcasted_iota(jnp.int32, (B, 1), 0)
    onehot = (col == targets).astype(jnp.float32)
    last = _dg(onehot, final)
    q = _dgT(last, wq_ref[...]) + bq_ref[...]
    kk = _dgT(final, wk_ref[...]) + bk_ref[...]
    vv = _dgT(final, wv_ref[...]) + bv_ref[...]
    scale = float(1.0 / np.sqrt(HD))
    parts = []
    for h in range(NH):
        hs = slice(h * HD, (h + 1) * HD)
        lg = _dgT(q[:, hs], kk[:, hs]) * scale
        lg = jnp.where(valid, lg, NEG)
        m = jnp.max(lg, axis=1, keepdims=True)
        e = jnp.exp(lg - m)
        a = e / jnp.sum(e, axis=1, keepdims=True)
        parts.append(_dg(a, vv[:, hs]))
    ao = jnp.concatenate(parts, axis=1)
    s_global = _dgT(ao, wo_ref[...]) + bo_ref[...]
    rv_ref[...] = _dgT(last, w3b_ref[...]) + _dgT(s_global, w3c_ref[...])


def _score_body(cand_ref, fin_ref, finw_ref, rv_ref, items_ref, w3a_ref,
                out_ref):
    cand = cand_ref[...]
    candp = _dg(cand, w3a_ref[...])
    base = _dgT(rv_ref[...], cand)
    rows = []
    for b in range(B):
        mask = items_ref[b] == 0
        ts = _dgT(finw_ref[pl.ds(b * LP, LP)], cand)
        ts = jnp.where(mask, NEG, ts)
        m = jnp.max(ts, axis=0, keepdims=True)
        e = jnp.exp(ts - m)
        fp = _dgT(fin_ref[pl.ds(b * LP, LP)], candp)
        num = jnp.sum(e * fp, axis=0, keepdims=True)
        den = jnp.sum(e, axis=0, keepdims=True)
        rows.append(num / den + base[b:b + 1, :])
    out_ref[...] = jnp.concatenate(rows, axis=0)


def kernel(session_items, session_len, session_adj, global_adj, emb, Wg, bg,
           w_ih, w_hh, b_ih, b_hh, W_ein, b_ein, W_eout, b_eout, pos_emb,
           in_proj_w, in_proj_b, out_proj_w, out_proj_b, W_target, W_3):
    session_items = session_items.astype(jnp.int32)
    session_len = session_len.astype(jnp.int32)
    f32 = jnp.float32

    items_p = jnp.pad(session_items, ((0, 0), (0, LP - L)))
    idxp = items_p.reshape(-1)
    positions = jnp.arange(L, dtype=jnp.int32)[None, :]
    rev = session_len[:, None] - 1 - positions
    rev = jnp.where(session_items == 0, 0, rev)
    rev = jnp.clip(rev, 0, 199)
    revp = jnp.pad(rev, ((0, 0), (0, LP - L))).reshape(-1)
    sadj_p = jnp.pad(session_adj, ((0, 0), (0, LP - L), (0, LP - L)))
    items_col = items_p.reshape(B, LP, 1)
    last_idx = jnp.clip(session_len - 1, 0, L - 1)

    hid_flat, pos_flat = _sc_gather(
        emb.astype(f32), pos_emb.astype(f32), idxp, revp)

    sgf = pl.pallas_call(
        _spmm_body,
        grid_spec=pltpu.PrefetchScalarGridSpec(
            num_scalar_prefetch=1,
            grid=(BLP // GRB,),
            in_specs=[
                pl.BlockSpec(memory_space=pltpu.MemorySpace.HBM),
                pl.BlockSpec((N, H), lambda i, idx: (0, 0)),
                pl.BlockSpec((H, H), lambda i, idx: (0, 0)),
                pl.BlockSpec((1, H), lambda i, idx: (0, 0)),
            ],
            out_specs=pl.BlockSpec((GRB, H), lambda i, idx: (i, 0)),
            scratch_shapes=[
                pltpu.VMEM((2, GRB, N), f32),
                pltpu.SemaphoreType.DMA((2,)),
            ],
        ),
        out_shape=jax.ShapeDtypeStruct((BLP, H), f32),
    )(idxp, global_adj, emb, Wg, bg.reshape(1, H))

    Wq, Wk, Wv = in_proj_w[:H], in_proj_w[H:2 * H], in_proj_w[2 * H:]
    bq = in_proj_b[:H].reshape(1, H)
    bk = in_proj_b[H:2 * H].reshape(1, H)
    bv = in_proj_b[2 * H:].reshape(1, H)
    W3a, W3b, W3c = W_3[:, :H], W_3[:, H:2 * H], W_3[:, 2 * H:]

    full = lambda shp: pl.BlockSpec(shp, lambda: tuple(0 for _ in shp))
    fin, finw, rv = pl.pallas_call(
        _sess_body,
        in_specs=[
            full((BLP, H)), full((BLP, H)), full((BLP, H)),
            full((B, LP, LP)), full((1, BLP)), full((B, 1)),
            full((H, H)), full((1, H)), full((H, H)), full((1, H)),
            full((3 * H, 2 * H)), full((3 * H, H)),
            full((1, 3 * H)), full((1, 3 * H)),
            full((H, H)), full((H, H)), full((H, H)),
            full((1, H)), full((1, H)), full((1, H)),
            full((H, H)), full((1, H)), full((H, H)),
            full((H, H)), full((H, H)),
        ],
        out_specs=[
            full((BLP, H)), full((BLP, H)), full((B, H)),
        ],
        out_shape=[
            jax.ShapeDtypeStruct((BLP, H), f32),
            jax.ShapeDtypeStruct((BLP, H), f32),
            jax.ShapeDtypeStruct((B, H), f32),
        ],
    )(sgf, hid_flat, pos_flat, sadj_p, items_p.reshape(1, BLP),
      last_idx.reshape(B, 1),
      W_ein, b_ein.reshape(1, H), W_eout, b_eout.reshape(1, H),
      w_ih, w_hh, b_ih.reshape(1, 3 * H), b_hh.reshape(1, 3 * H),
      Wq, Wk, Wv, bq, bk, bv,
      out_proj_w, out_proj_b.reshape(1, H), W_target, W3b, W3c)

    scores_full = pl.pallas_call(
        _score_body,
        grid=(MP // MBLK,),
        in_specs=[
            pl.BlockSpec((MBLK, H), lambda i: (i, 0)),
            pl.BlockSpec((BLP, H), lambda i: (0, 0)),
            pl.BlockSpec((BLP, H), lambda i: (0, 0)),
            pl.BlockSpec((B, H), lambda i: (0, 0)),
            pl.BlockSpec((B, LP, 1), lambda i: (0, 0, 0)),
            pl.BlockSpec((H, H), lambda i: (0, 0)),
        ],
        out_specs=pl.BlockSpec((B, MBLK), lambda i: (0, i)),
        out_shape=jax.ShapeDtypeStruct((B, N), f32),
    )(emb, fin, finw, rv, items_col, W3a)

    return scores_full[:, 1:]


# speedup vs baseline: 3.0691x; 1.0337x over previous
import functools

import jax
import jax.numpy as jnp
import numpy as np
from jax import lax
from jax.experimental import pallas as pl
from jax.experimental.pallas import tpu as pltpu
from jax.experimental.pallas import tpu_sc as plsc

B, L, N, H, NH = 16, 50, 10000, 128, 4
HD = H // NH
LP = 56
BLP = B * LP
RPW = 32
NSCW = BLP // RPW
GRB = 112
MBLK = 1024
MP = 10240
NEG = -1e30

_sc_mesh = plsc.VectorSubcoreMesh(core_axis_name="c", subcore_axis_name="s")


@functools.partial(
    pl.kernel,
    out_type=(
        jax.ShapeDtypeStruct((BLP, H), jnp.float32),
        jax.ShapeDtypeStruct((BLP, H), jnp.float32),
    ),
    mesh=_sc_mesh,
    scratch_types=[
        pltpu.VMEM((RPW,), jnp.int32),
        pltpu.VMEM((RPW,), jnp.int32),
        pltpu.VMEM((RPW, H), jnp.float32),
        pltpu.VMEM((RPW, H), jnp.float32),
        pltpu.SemaphoreType.DMA,
        pltpu.SemaphoreType.DMA,
    ],
)
def _sc_gather(emb_hbm, pos_hbm, idx_hbm, rev_hbm,
               emb_out, pos_out, idx_v, rev_v, erow_v, prow_v, sem, sem2):
    wid = lax.axis_index("s") * 2 + lax.axis_index("c")

    @pl.when(wid < NSCW)
    def _():
        b = wid * RPW
        pltpu.sync_copy(idx_hbm.at[pl.ds(b, RPW)], idx_v)
        pltpu.sync_copy(rev_hbm.at[pl.ds(b, RPW)], rev_v)
        ce = pltpu.async_copy(emb_hbm.at[idx_v], erow_v, sem)
        cp = pltpu.async_copy(pos_hbm.at[rev_v], prow_v, sem2)
        ce.wait()
        cp.wait()
        pltpu.sync_copy(erow_v, emb_out.at[pl.ds(b, RPW)])
        pltpu.sync_copy(prow_v, pos_out.at[pl.ds(b, RPW)])


def _dgT(x, w):
    return lax.dot_general(x, w, (((1,), (1,)), ((), ())),
                           preferred_element_type=jnp.float32)


def _dg(x, w):
    return lax.dot_general(x, w, (((1,), (0,)), ((), ())),
                           preferred_element_type=jnp.float32)


def _spmm_body(idx_ref, adj_any, emb_ref, wg_ref, bg_ref, out_ref,
               rows_v, sems):
    i = pl.program_id(0)
    nb = pl.num_programs(0)

    def issue(block, slot):
        for r in range(GRB):
            if r % LP < L:
                pltpu.make_async_copy(
                    adj_any.at[pl.ds(idx_ref[block * GRB + r], 1), :],
                    rows_v.at[slot, pl.ds(r, 1), :], sems.at[slot]).start()

    def drain_compute(slot):
        for r in range(GRB):
            if r % LP < L:
                pltpu.make_async_copy(
                    adj_any.at[pl.ds(0, 1), :],
                    rows_v.at[slot, pl.ds(r, 1), :], sems.at[slot]).wait()
        g = jnp.dot(rows_v[slot], emb_ref[...],
                    preferred_element_type=jnp.float32)
        val = jnp.maximum(_dgT(g, wg_ref[...]) + bg_ref[...], 0.0)
        rid = lax.broadcasted_iota(jnp.int32, (GRB, 1), 0)
        out_ref[...] = jnp.where(rid % LP < L, val, 0.0)

    @pl.when(i == 0)
    def _():
        issue(0, 0)

    p = lax.rem(i, 2)

    @pl.when(p == 0)
    def _():
        @pl.when(i + 1 < nb)
        def _():
            issue(i + 1, 1)
        drain_compute(0)

    @pl.when(p == 1)
    def _():
        @pl.when(i + 1 < nb)
        def _():
            issue(i + 1, 0)
        drain_compute(1)


def _sess_body(sgl_ref, hid_ref, pos_ref, sadj_ref, items_ref, lidx_ref,
               wein_ref, bein_ref, weout_ref, beout_ref, wih_ref, whh_ref,
               bih_ref, bhh_ref, wq_ref, wk_ref, wv_ref, bq_ref, bk_ref,
               bv_ref, wo_ref, bo_ref, wt_ref, w3b_ref, w3c_ref,
               fin_ref, finw_ref, rv_ref):
    hid = hid_ref[...]
    ein = _dgT(hid, wein_ref[...]) + bein_ref[...]
    eout = _dgT(hid, weout_ref[...]) + beout_ref[...]
    iis, ios = [], []
    for b in range(B):
        A = sadj_ref[b]
        iis.append(_dg(A, ein[b * LP:(b + 1) * LP]))
        ios.append(_dg(A, eout[b * LP:(b + 1) * LP]))
    inputs = jnp.concatenate(
        [jnp.concatenate(iis, axis=0), jnp.concatenate(ios, axis=0)], axis=1)
    gi = _dgT(inputs, wih_ref[...]) + bih_ref[...]
    gh = _dgT(hid, whh_ref[...]) + bhh_ref[...]
    i_r, i_i, i_n = gi[:, :H], gi[:, H:2 * H], gi[:, 2 * H:]
    h_r, h_i, h_n = gh[:, :H], gh[:, H:2 * H], gh[:, 2 * H:]
    rg = jax.nn.sigmoid(i_r + h_r)
    ig = jax.nn.sigmoid(i_i + h_i)
    ng = jnp.tanh(i_n + rg * h_n)
    hy = ng + ig * (hid - ng)
    final = sgl_ref[...] + hy + pos_ref[...]
    fin_ref[...] = final
    finw_ref[...] = _dg(final, wt_ref[...])
    rowid = lax.broadcasted_iota(jnp.int32, (B, BLP), 0)
    col = lax.broadcasted_iota(jnp.int32, (B, BLP), 1)
    valid = (col // LP == rowid) & (items_ref[...] != 0)
    targets = lidx_ref[...] + LP * lax.broadcasted_iota(jnp.int32, (B, 1), 0)
    onehot = (col == targets).astype(jnp.float32)
    last = _dg(onehot, final)
    q = _dgT(last, wq_ref[...]) + bq_ref[...]
    kk = _dgT(final, wk_ref[...]) + bk_ref[...]
    vv = _dgT(final, wv_ref[...]) + bv_ref[...]
    scale = float(1.0 / np.sqrt(HD))
    parts = []
    for h in range(NH):
        hs = slice(h * HD, (h + 1) * HD)
        lg = _dgT(q[:, hs], kk[:, hs]) * scale
        lg = jnp.where(valid, lg, NEG)
        m = jnp.max(lg, axis=1, keepdims=True)
        e = jnp.exp(lg - m)
        a = e / jnp.sum(e, axis=1, keepdims=True)
        parts.append(_dg(a, vv[:, hs]))
    ao = jnp.concatenate(parts, axis=1)
    s_global = _dgT(ao, wo_ref[...]) + bo_ref[...]
    rv_ref[...] = _dgT(last, w3b_ref[...]) + _dgT(s_global, w3c_ref[...])


def _score_body(cand_ref, fin_ref, finw_ref, rv_ref, items_ref, w3a_ref,
                out_ref):
    cand = cand_ref[...]
    candp = _dg(cand, w3a_ref[...])
    base = _dgT(rv_ref[...], cand)
    rows = []
    for b in range(B):
        mask = items_ref[b] == 0
        ts = _dgT(finw_ref[pl.ds(b * LP, LP)], cand)
        ts = jnp.where(mask, NEG, ts)
        m = jnp.max(ts, axis=0, keepdims=True)
        e = jnp.exp(ts - m)
        fp = _dgT(fin_ref[pl.ds(b * LP, LP)], candp)
        num = jnp.sum(e * fp, axis=0, keepdims=True)
        den = jnp.sum(e, axis=0, keepdims=True)
        rows.append(num / den + base[b:b + 1, :])
    out_ref[...] = jnp.concatenate(rows, axis=0)


def kernel(session_items, session_len, session_adj, global_adj, emb, Wg, bg,
           w_ih, w_hh, b_ih, b_hh, W_ein, b_ein, W_eout, b_eout, pos_emb,
           in_proj_w, in_proj_b, out_proj_w, out_proj_b, W_target, W_3):
    session_items = session_items.astype(jnp.int32)
    session_len = session_len.astype(jnp.int32)
    f32 = jnp.float32

    items_p = jnp.pad(session_items, ((0, 0), (0, LP - L)))
    idxp = items_p.reshape(-1)
    positions = jnp.arange(L, dtype=jnp.int32)[None, :]
    rev = session_len[:, None] - 1 - positions
    rev = jnp.where(session_items == 0, 0, rev)
    rev = jnp.clip(rev, 0, 199)
    revp = jnp.pad(rev, ((0, 0), (0, LP - L))).reshape(-1)
    sadj_p = jnp.pad(session_adj, ((0, 0), (0, LP - L), (0, LP - L)))
    items_col = items_p.reshape(B, LP, 1)
    last_idx = jnp.clip(session_len - 1, 0, L - 1)

    hid_flat, pos_flat = _sc_gather(
        emb.astype(f32), pos_emb.astype(f32), idxp, revp)

    sgf = pl.pallas_call(
        _spmm_body,
        grid_spec=pltpu.PrefetchScalarGridSpec(
            num_scalar_prefetch=1,
            grid=(BLP // GRB,),
            in_specs=[
                pl.BlockSpec(memory_space=pltpu.MemorySpace.HBM),
                pl.BlockSpec((N, H), lambda i, idx: (0, 0)),
                pl.BlockSpec((H, H), lambda i, idx: (0, 0)),
                pl.BlockSpec((1, H), lambda i, idx: (0, 0)),
            ],
            out_specs=pl.BlockSpec((GRB, H), lambda i, idx: (i, 0)),
            scratch_shapes=[
                pltpu.VMEM((2, GRB, N), f32),
                pltpu.SemaphoreType.DMA((2,)),
            ],
        ),
        out_shape=jax.ShapeDtypeStruct((BLP, H), f32),
    )(idxp, global_adj, emb, Wg, bg.reshape(1, H))

    Wq, Wk, Wv = in_proj_w[:H], in_proj_w[H:2 * H], in_proj_w[2 * H:]
    bq = in_proj_b[:H].reshape(1, H)
    bk = in_proj_b[H:2 * H].reshape(1, H)
    bv = in_proj_b[2 * H:].reshape(1, H)
    W3a, W3b, W3c = W_3[:, :H], W_3[:, H:2 * H], W_3[:, 2 * H:]

    full = lambda shp: pl.BlockSpec(shp, lambda: tuple(0 for _ in shp))
    fin, finw, rv = pl.pallas_call(
        _sess_body,
        in_specs=[
            full((BLP, H)), full((BLP, H)), full((BLP, H)),
            full((B, LP, LP)), full((1, BLP)), full((B, 1)),
            full((H, H)), full((1, H)), full((H, H)), full((1, H)),
            full((3 * H, 2 * H)), full((3 * H, H)),
            full((1, 3 * H)), full((1, 3 * H)),
            full((H, H)), full((H, H)), full((H, H)),
            full((1, H)), full((1, H)), full((1, H)),
            full((H, H)), full((1, H)), full((H, H)),
            full((H, H)), full((H, H)),
        ],
        out_specs=[
            full((BLP, H)), full((BLP, H)), full((B, H)),
        ],
        out_shape=[
            jax.ShapeDtypeStruct((BLP, H), f32),
            jax.ShapeDtypeStruct((BLP, H), f32),
            jax.ShapeDtypeStruct((B, H), f32),
        ],
    )(sgf, hid_flat, pos_flat, sadj_p, items_p.reshape(1, BLP),
      last_idx.reshape(B, 1),
      W_ein, b_ein.reshape(1, H), W_eout, b_eout.reshape(1, H),
      w_ih, w_hh, b_ih.reshape(1, 3 * H), b_hh.reshape(1, 3 * H),
      Wq, Wk, Wv, bq, bk, bv,
      out_proj_w, out_proj_b.reshape(1, H), W_target, W3b, W3c)

    scores_full = pl.pallas_call(
        _score_body,
        grid=(MP // MBLK,),
        in_specs=[
            pl.BlockSpec((MBLK, H), lambda i: (i, 0)),
            pl.BlockSpec((BLP, H), lambda i: (0, 0)),
            pl.BlockSpec((BLP, H), lambda i: (0, 0)),
            pl.BlockSpec((B, H), lambda i: (0, 0)),
            pl.BlockSpec((B, LP, 1), lambda i: (0, 0, 0)),
            pl.BlockSpec((H, H), lambda i: (0, 0)),
        ],
        out_specs=pl.BlockSpec((B, MBLK), lambda i: (0, i)),
        out_shape=jax.ShapeDtypeStruct((B, N), f32),
    )(emb, fin, finw, rv, items_col, W3a)

    return scores_full[:, 1:]


# speedup vs baseline: 3.2776x; 1.0679x over previous
import functools

import jax
import jax.numpy as jnp
import numpy as np
from jax import lax
from jax.experimental import pallas as pl
from jax.experimental.pallas import tpu as pltpu
from jax.experimental.pallas import tpu_sc as plsc

B, L, N, H, NH = 16, 50, 10000, 128, 4
HD = H // NH
LP = 56
BLP = B * LP
RPW = 32
NSCW = BLP // RPW
GRB = 112
MBLK = 2048
MP = 10240
NEG = -1e30

_sc_mesh = plsc.VectorSubcoreMesh(core_axis_name="c", subcore_axis_name="s")


@functools.partial(
    pl.kernel,
    out_type=(
        jax.ShapeDtypeStruct((BLP, H), jnp.float32),
        jax.ShapeDtypeStruct((BLP, H), jnp.float32),
    ),
    mesh=_sc_mesh,
    scratch_types=[
        pltpu.VMEM((RPW,), jnp.int32),
        pltpu.VMEM((RPW,), jnp.int32),
        pltpu.VMEM((RPW, H), jnp.float32),
        pltpu.VMEM((RPW, H), jnp.float32),
        pltpu.SemaphoreType.DMA,
        pltpu.SemaphoreType.DMA,
    ],
)
def _sc_gather(emb_hbm, pos_hbm, idx_hbm, rev_hbm,
               emb_out, pos_out, idx_v, rev_v, erow_v, prow_v, sem, sem2):
    wid = lax.axis_index("s") * 2 + lax.axis_index("c")

    @pl.when(wid < NSCW)
    def _():
        b = wid * RPW
        pltpu.sync_copy(idx_hbm.at[pl.ds(b, RPW)], idx_v)
        pltpu.sync_copy(rev_hbm.at[pl.ds(b, RPW)], rev_v)
        ce = pltpu.async_copy(emb_hbm.at[idx_v], erow_v, sem)
        cp = pltpu.async_copy(pos_hbm.at[rev_v], prow_v, sem2)
        ce.wait()
        cp.wait()
        pltpu.sync_copy(erow_v, emb_out.at[pl.ds(b, RPW)])
        pltpu.sync_copy(prow_v, pos_out.at[pl.ds(b, RPW)])


def _dgT(x, w):
    return lax.dot_general(x, w, (((1,), (1,)), ((), ())),
                           preferred_element_type=jnp.float32)


def _dg(x, w):
    return lax.dot_general(x, w, (((1,), (0,)), ((), ())),
                           preferred_element_type=jnp.float32)


def _spmm_body(idx_ref, adj_any, emb_ref, wg_ref, bg_ref, out_ref,
               rows_v, sems):
    i = pl.program_id(0)
    nb = pl.num_programs(0)

    def issue(block, slot):
        for r in range(GRB):
            if r % LP < L:
                pltpu.make_async_copy(
                    adj_any.at[pl.ds(idx_ref[block * GRB + r], 1), :],
                    rows_v.at[slot, pl.ds(r, 1), :],
                    sems.at[slot, r % 4]).start()

    def drain_compute(slot):
        for r in range(GRB):
            if r % LP < L:
                pltpu.make_async_copy(
                    adj_any.at[pl.ds(0, 1), :],
                    rows_v.at[slot, pl.ds(r, 1), :],
                    sems.at[slot, r % 4]).wait()
        g = jnp.dot(rows_v[slot], emb_ref[...],
                    preferred_element_type=jnp.float32)
        val = jnp.maximum(_dgT(g, wg_ref[...]) + bg_ref[...], 0.0)
        rid = lax.broadcasted_iota(jnp.int32, (GRB, 1), 0)
        out_ref[...] = jnp.where(rid % LP < L, val, 0.0)

    @pl.when(i == 0)
    def _():
        issue(0, 0)

    p = lax.rem(i, 2)

    @pl.when(p == 0)
    def _():
        @pl.when(i + 1 < nb)
        def _():
            issue(i + 1, 1)
        drain_compute(0)

    @pl.when(p == 1)
    def _():
        @pl.when(i + 1 < nb)
        def _():
            issue(i + 1, 0)
        drain_compute(1)


def _sess_body(sgl_ref, hid_ref, pos_ref, sadj_ref, items_ref, lidx_ref,
               wein_ref, bein_ref, weout_ref, beout_ref, wih_ref, whh_ref,
               bih_ref, bhh_ref, wq_ref, wk_ref, wv_ref, bq_ref, bk_ref,
               bv_ref, wo_ref, bo_ref, wt_ref, w3b_ref, w3c_ref,
               fin_ref, finw_ref, rv_ref):
    hid = hid_ref[...]
    ein = _dgT(hid, wein_ref[...]) + bein_ref[...]
    eout = _dgT(hid, weout_ref[...]) + beout_ref[...]
    iis, ios = [], []
    for b in range(B):
        A = sadj_ref[b]
        iis.append(_dg(A, ein[b * LP:(b + 1) * LP]))
        ios.append(_dg(A, eout[b * LP:(b + 1) * LP]))
    inputs = jnp.concatenate(
        [jnp.concatenate(iis, axis=0), jnp.concatenate(ios, axis=0)], axis=1)
    gi = _dgT(inputs, wih_ref[...]) + bih_ref[...]
    gh = _dgT(hid, whh_ref[...]) + bhh_ref[...]
    i_r, i_i, i_n = gi[:, :H], gi[:, H:2 * H], gi[:, 2 * H:]
    h_r, h_i, h_n = gh[:, :H], gh[:, H:2 * H], gh[:, 2 * H:]
    rg = jax.nn.sigmoid(i_r + h_r)
    ig = jax.nn.sigmoid(i_i + h_i)
    ng = jnp.tanh(i_n + rg * h_n)
    hy = ng + ig * (hid - ng)
    final = sgl_ref[...] + hy + pos_ref[...]
    fin_ref[...] = final
    finw_ref[...] = _dg(final, wt_ref[...])
    rowid = lax.broadcasted_iota(jnp.int32, (B, BLP), 0)
    col = lax.broadcasted_iota(jnp.int32, (B, BLP), 1)
    valid = (col // LP == rowid) & (items_ref[...] != 0)
    targets = lidx_ref[...] + LP * lax.broadcasted_iota(jnp.int32, (B, 1), 0)
    onehot = (col == targets).astype(jnp.float32)
    last = _dg(onehot, final)
    q = _dgT(last, wq_ref[...]) + bq_ref[...]
    kk = _dgT(final, wk_ref[...]) + bk_ref[...]
    vv = _dgT(final, wv_ref[...]) + bv_ref[...]
    scale = float(1.0 / np.sqrt(HD))
    parts = []
    for h in range(NH):
        hs = slice(h * HD, (h + 1) * HD)
        lg = _dgT(q[:, hs], kk[:, hs]) * scale
        lg = jnp.where(valid, lg, NEG)
        m = jnp.max(lg, axis=1, keepdims=True)
        e = jnp.exp(lg - m)
        a = e / jnp.sum(e, axis=1, keepdims=True)
        parts.append(_dg(a, vv[:, hs]))
    ao = jnp.concatenate(parts, axis=1)
    s_global = _dgT(ao, wo_ref[...]) + bo_ref[...]
    rv_ref[...] = _dgT(last, w3b_ref[...]) + _dgT(s_global, w3c_ref[...])


def _score_body(cand_ref, fin_ref, finw_ref, rv_ref, items_ref, w3a_ref,
                out_ref):
    cand = cand_ref[...]
    candp = _dg(cand, w3a_ref[...])
    base = _dgT(rv_ref[...], cand)
    rows = []
    for b in range(B):
        mask = items_ref[b] == 0
        ts = _dgT(finw_ref[pl.ds(b * LP, LP)], cand)
        ts = jnp.where(mask, NEG, ts)
        m = jnp.max(ts, axis=0, keepdims=True)
        e = jnp.exp(ts - m)
        fp = _dgT(fin_ref[pl.ds(b * LP, LP)], candp)
        num = jnp.sum(e * fp, axis=0, keepdims=True)
        den = jnp.sum(e, axis=0, keepdims=True)
        rows.append(num / den + base[b:b + 1, :])
    out_ref[...] = jnp.concatenate(rows, axis=0)


def kernel(session_items, session_len, session_adj, global_adj, emb, Wg, bg,
           w_ih, w_hh, b_ih, b_hh, W_ein, b_ein, W_eout, b_eout, pos_emb,
           in_proj_w, in_proj_b, out_proj_w, out_proj_b, W_target, W_3):
    session_items = session_items.astype(jnp.int32)
    session_len = session_len.astype(jnp.int32)
    f32 = jnp.float32

    items_p = jnp.pad(session_items, ((0, 0), (0, LP - L)))
    idxp = items_p.reshape(-1)
    positions = jnp.arange(L, dtype=jnp.int32)[None, :]
    rev = session_len[:, None] - 1 - positions
    rev = jnp.where(session_items == 0, 0, rev)
    rev = jnp.clip(rev, 0, 199)
    revp = jnp.pad(rev, ((0, 0), (0, LP - L))).reshape(-1)
    sadj_p = jnp.pad(session_adj, ((0, 0), (0, LP - L), (0, LP - L)))
    items_col = items_p.reshape(B, LP, 1)
    last_idx = jnp.clip(session_len - 1, 0, L - 1)

    hid_flat, pos_flat = _sc_gather(
        emb.astype(f32), pos_emb.astype(f32), idxp, revp)

    sgf = pl.pallas_call(
        _spmm_body,
        grid_spec=pltpu.PrefetchScalarGridSpec(
            num_scalar_prefetch=1,
            grid=(BLP // GRB,),
            in_specs=[
                pl.BlockSpec(memory_space=pltpu.MemorySpace.HBM),
                pl.BlockSpec((N, H), lambda i, idx: (0, 0)),
                pl.BlockSpec((H, H), lambda i, idx: (0, 0)),
                pl.BlockSpec((1, H), lambda i, idx: (0, 0)),
            ],
            out_specs=pl.BlockSpec((GRB, H), lambda i, idx: (i, 0)),
            scratch_shapes=[
                pltpu.VMEM((2, GRB, N), f32),
                pltpu.SemaphoreType.DMA((2, 4)),
            ],
        ),
        out_shape=jax.ShapeDtypeStruct((BLP, H), f32),
    )(idxp, global_adj, emb, Wg, bg.reshape(1, H))

    Wq, Wk, Wv = in_proj_w[:H], in_proj_w[H:2 * H], in_proj_w[2 * H:]
    bq = in_proj_b[:H].reshape(1, H)
    bk = in_proj_b[H:2 * H].reshape(1, H)
    bv = in_proj_b[2 * H:].reshape(1, H)
    W3a, W3b, W3c = W_3[:, :H], W_3[:, H:2 * H], W_3[:, 2 * H:]

    full = lambda shp: pl.BlockSpec(shp, lambda: tuple(0 for _ in shp))
    fin, finw, rv = pl.pallas_call(
        _sess_body,
        in_specs=[
            full((BLP, H)), full((BLP, H)), full((BLP, H)),
            full((B, LP, LP)), full((1, BLP)), full((B, 1)),
            full((H, H)), full((1, H)), full((H, H)), full((1, H)),
            full((3 * H, 2 * H)), full((3 * H, H)),
            full((1, 3 * H)), full((1, 3 * H)),
            full((H, H)), full((H, H)), full((H, H)),
            full((1, H)), full((1, H)), full((1, H)),
            full((H, H)), full((1, H)), full((H, H)),
            full((H, H)), full((H, H)),
        ],
        out_specs=[
            full((BLP, H)), full((BLP, H)), full((B, H)),
        ],
        out_shape=[
            jax.ShapeDtypeStruct((BLP, H), f32),
            jax.ShapeDtypeStruct((BLP, H), f32),
            jax.ShapeDtypeStruct((B, H), f32),
        ],
    )(sgf, hid_flat, pos_flat, sadj_p, items_p.reshape(1, BLP),
      last_idx.reshape(B, 1),
      W_ein, b_ein.reshape(1, H), W_eout, b_eout.reshape(1, H),
      w_ih, w_hh, b_ih.reshape(1, 3 * H), b_hh.reshape(1, 3 * H),
      Wq, Wk, Wv, bq, bk, bv,
      out_proj_w, out_proj_b.reshape(1, H), W_target, W3b, W3c)

    scores_full = pl.pallas_call(
        _score_body,
        grid=(MP // MBLK,),
        in_specs=[
            pl.BlockSpec((MBLK, H), lambda i: (i, 0)),
            pl.BlockSpec((BLP, H), lambda i: (0, 0)),
            pl.BlockSpec((BLP, H), lambda i: (0, 0)),
            pl.BlockSpec((B, H), lambda i: (0, 0)),
            pl.BlockSpec((B, LP, 1), lambda i: (0, 0, 0)),
            pl.BlockSpec((H, H), lambda i: (0, 0)),
        ],
        out_specs=pl.BlockSpec((B, MBLK), lambda i: (0, i)),
        out_shape=jax.ShapeDtypeStruct((B, N), f32),
    )(emb, fin, finw, rv, items_col, W3a)

    return scores_full[:, 1:]


# speedup vs baseline: 4.0072x; 1.2226x over previous
import functools

import jax
import jax.numpy as jnp
import numpy as np
from jax import lax
from jax.experimental import pallas as pl
from jax.experimental.pallas import tpu as pltpu
from jax.experimental.pallas import tpu_sc as plsc

B, L, N, H, NH = 16, 50, 10000, 128, 4
HD = H // NH
LP = 56
BLP = B * LP
RPW = 32
NSCW = BLP // RPW
GRB = 112
MBLK = 2048
MP = 10240
NEG = -1e30

_sc_mesh = plsc.VectorSubcoreMesh(core_axis_name="c", subcore_axis_name="s")


@functools.partial(
    pl.kernel,
    out_type=jax.ShapeDtypeStruct((BLP, H), jnp.float32),
    mesh=_sc_mesh,
    scratch_types=[
        pltpu.VMEM((RPW,), jnp.int32),
        pltpu.VMEM((RPW, H), jnp.float32),
        pltpu.SemaphoreType.DMA,
    ],
)
def _sc_gather(emb_hbm, idx_hbm, emb_out, idx_v, erow_v, sem):
    wid = lax.axis_index("s") * 2 + lax.axis_index("c")

    @pl.when(wid < NSCW)
    def _():
        b = wid * RPW
        pltpu.sync_copy(idx_hbm.at[pl.ds(b, RPW)], idx_v)
        pltpu.async_copy(emb_hbm.at[idx_v], erow_v, sem).wait()
        pltpu.sync_copy(erow_v, emb_out.at[pl.ds(b, RPW)])


def _dgT(x, w):
    return lax.dot_general(x, w, (((1,), (1,)), ((), ())),
                           preferred_element_type=jnp.float32)


def _dg(x, w):
    return lax.dot_general(x, w, (((1,), (0,)), ((), ())),
                           preferred_element_type=jnp.float32)


def _spmm_body(idx_ref, adj_any, emb_ref, wg_ref, bg_ref, out_ref,
               rows_v, sems):
    i = pl.program_id(0)
    nb = pl.num_programs(0)

    def issue(block, slot):
        for r in range(GRB):
            if r % LP < L:
                pltpu.make_async_copy(
                    adj_any.at[pl.ds(idx_ref[block * GRB + r], 1), :],
                    rows_v.at[slot, pl.ds(r, 1), :],
                    sems.at[slot, r % 4]).start()

    def drain_compute(slot):
        for r in range(GRB):
            if r % LP < L:
                pltpu.make_async_copy(
                    adj_any.at[pl.ds(0, 1), :],
                    rows_v.at[slot, pl.ds(r, 1), :],
                    sems.at[slot, r % 4]).wait()
        g = jnp.dot(rows_v[slot], emb_ref[...],
                    preferred_element_type=jnp.float32)
        val = jnp.maximum(_dgT(g, wg_ref[...]) + bg_ref[...], 0.0)
        rid = lax.broadcasted_iota(jnp.int32, (GRB, 1), 0)
        out_ref[...] = jnp.where(rid % LP < L, val, 0.0)

    @pl.when(i == 0)
    def _():
        issue(0, 0)

    p = lax.rem(i, 2)

    @pl.when(p == 0)
    def _():
        @pl.when(i + 1 < nb)
        def _():
            issue(i + 1, 1)
        drain_compute(0)

    @pl.when(p == 1)
    def _():
        @pl.when(i + 1 < nb)
        def _():
            issue(i + 1, 0)
        drain_compute(1)


def _sess_body(sgl_ref, hid_ref, pemb_ref, sadj_ref, items_ref, icol_ref,
               len_ref,
               wein_ref, bein_ref, weout_ref, beout_ref, wih_ref, whh_ref,
               bih_ref, bhh_ref, wq_ref, wk_ref, wv_ref, bq_ref, bk_ref,
               bv_ref, wo_ref, bo_ref, wt_ref, w3b_ref, w3c_ref,
               fin_ref, finw_ref, rv_ref):
    hid = hid_ref[...]
    icol = icol_ref[...].reshape(BLP, 1)
    lcol = (lax.broadcasted_iota(jnp.int32, (BLP, 1), 0) %
            LP).astype(jnp.float32)
    ssel = (lax.broadcasted_iota(jnp.int32, (BLP, B), 0) // LP ==
            lax.broadcasted_iota(jnp.int32, (BLP, B), 1)).astype(jnp.float32)
    lencol = _dg(ssel, len_ref[...].astype(jnp.float32))
    revc = jnp.where(icol == 0, 0.0, lencol - 1.0 - lcol)
    revc = jnp.clip(revc, 0.0, 199.0).astype(jnp.int32)
    oh = (lax.broadcasted_iota(jnp.int32, (BLP, 200), 1) ==
          revc).astype(jnp.float32)
    pos = _dg(oh, pemb_ref[...])
    ein = _dgT(hid, wein_ref[...]) + bein_ref[...]
    eout = _dgT(hid, weout_ref[...]) + beout_ref[...]
    iis, ios = [], []
    for b in range(B):
        A = sadj_ref[b]
        iis.append(_dg(A, ein[b * LP:(b + 1) * LP]))
        ios.append(_dg(A, eout[b * LP:(b + 1) * LP]))
    inputs = jnp.concatenate(
        [jnp.concatenate(iis, axis=0), jnp.concatenate(ios, axis=0)], axis=1)
    gi = _dgT(inputs, wih_ref[...]) + bih_ref[...]
    gh = _dgT(hid, whh_ref[...]) + bhh_ref[...]
    i_r, i_i, i_n = gi[:, :H], gi[:, H:2 * H], gi[:, 2 * H:]
    h_r, h_i, h_n = gh[:, :H], gh[:, H:2 * H], gh[:, 2 * H:]
    rg = jax.nn.sigmoid(i_r + h_r)
    ig = jax.nn.sigmoid(i_i + h_i)
    ng = jnp.tanh(i_n + rg * h_n)
    hy = ng + ig * (hid - ng)
    final = sgl_ref[...] + hy + pos
    fin_ref[...] = final
    finw_ref[...] = _dg(final, wt_ref[...])
    rowid = lax.broadcasted_iota(jnp.int32, (B, BLP), 0)
    col = lax.broadcasted_iota(jnp.int32, (B, BLP), 1)
    valid = (col // LP == rowid) & (items_ref[...] != 0)
    lidx = jnp.clip(len_ref[...] - 1, 0, L - 1)
    targets = lidx + LP * lax.broadcasted_iota(jnp.int32, (B, 1), 0)
    onehot = (col == targets).astype(jnp.float32)
    last = _dg(onehot, final)
    q = _dgT(last, wq_ref[...]) + bq_ref[...]
    kk = _dgT(final, wk_ref[...]) + bk_ref[...]
    vv = _dgT(final, wv_ref[...]) + bv_ref[...]
    scale = float(1.0 / np.sqrt(HD))
    parts = []
    for h in range(NH):
        hs = slice(h * HD, (h + 1) * HD)
        lg = _dgT(q[:, hs], kk[:, hs]) * scale
        lg = jnp.where(valid, lg, NEG)
        m = jnp.max(lg, axis=1, keepdims=True)
        e = jnp.exp(lg - m)
        a = e / jnp.sum(e, axis=1, keepdims=True)
        parts.append(_dg(a, vv[:, hs]))
    ao = jnp.concatenate(parts, axis=1)
    s_global = _dgT(ao, wo_ref[...]) + bo_ref[...]
    rv_ref[...] = _dgT(last, w3b_ref[...]) + _dgT(s_global, w3c_ref[...])


def _score_body(cand_ref, fin_ref, finw_ref, rv_ref, items_ref, w3a_ref,
                out_ref):
    cand = cand_ref[...]
    candp = _dg(cand, w3a_ref[...])
    base = _dgT(rv_ref[...], cand)
    rows = []
    for b in range(B):
        mask = items_ref[b] == 0
        ts = _dgT(finw_ref[pl.ds(b * LP, LP)], cand)
        ts = jnp.where(mask, NEG, ts)
        m = jnp.max(ts, axis=0, keepdims=True)
        e = jnp.exp(ts - m)
        fp = _dgT(fin_ref[pl.ds(b * LP, LP)], candp)
        num = jnp.sum(e * fp, axis=0, keepdims=True)
        den = jnp.sum(e, axis=0, keepdims=True)
        rows.append(num / den + base[b:b + 1, :])
    out_ref[...] = jnp.concatenate(rows, axis=0)


def kernel(session_items, session_len, session_adj, global_adj, emb, Wg, bg,
           w_ih, w_hh, b_ih, b_hh, W_ein, b_ein, W_eout, b_eout, pos_emb,
           in_proj_w, in_proj_b, out_proj_w, out_proj_b, W_target, W_3):
    session_items = session_items.astype(jnp.int32)
    session_len = session_len.astype(jnp.int32)
    f32 = jnp.float32

    items_p = jnp.pad(session_items, ((0, 0), (0, LP - L)))
    idxp = items_p.reshape(-1)
    sadj_p = jnp.pad(session_adj, ((0, 0), (0, LP - L), (0, LP - L)))
    items_col = items_p.reshape(B, LP, 1)

    hid_flat = _sc_gather(emb.astype(f32), idxp)

    sgf = pl.pallas_call(
        _spmm_body,
        grid_spec=pltpu.PrefetchScalarGridSpec(
            num_scalar_prefetch=1,
            grid=(BLP // GRB,),
            in_specs=[
                pl.BlockSpec(memory_space=pltpu.MemorySpace.HBM),
                pl.BlockSpec((N, H), lambda i, idx: (0, 0)),
                pl.BlockSpec((H, H), lambda i, idx: (0, 0)),
                pl.BlockSpec((1, H), lambda i, idx: (0, 0)),
            ],
            out_specs=pl.BlockSpec((GRB, H), lambda i, idx: (i, 0)),
            scratch_shapes=[
                pltpu.VMEM((2, GRB, N), f32),
                pltpu.SemaphoreType.DMA((2, 4)),
            ],
        ),
        out_shape=jax.ShapeDtypeStruct((BLP, H), f32),
    )(idxp, global_adj, emb, Wg, bg.reshape(1, H))

    Wq, Wk, Wv = in_proj_w[:H], in_proj_w[H:2 * H], in_proj_w[2 * H:]
    bq = in_proj_b[:H].reshape(1, H)
    bk = in_proj_b[H:2 * H].reshape(1, H)
    bv = in_proj_b[2 * H:].reshape(1, H)
    W3a, W3b, W3c = W_3[:, :H], W_3[:, H:2 * H], W_3[:, 2 * H:]

    full = lambda shp: pl.BlockSpec(shp, lambda: tuple(0 for _ in shp))
    fin, finw, rv = pl.pallas_call(
        _sess_body,
        in_specs=[
            full((BLP, H)), full((BLP, H)), full((200, H)),
            full((B, LP, LP)), full((1, BLP)), full((B, LP, 1)),
            full((B, 1)),
            full((H, H)), full((1, H)), full((H, H)), full((1, H)),
            full((3 * H, 2 * H)), full((3 * H, H)),
            full((1, 3 * H)), full((1, 3 * H)),
            full((H, H)), full((H, H)), full((H, H)),
            full((1, H)), full((1, H)), full((1, H)),
            full((H, H)), full((1, H)), full((H, H)),
            full((H, H)), full((H, H)),
        ],
        out_specs=[
            full((BLP, H)), full((BLP, H)), full((B, H)),
        ],
        out_shape=[
            jax.ShapeDtypeStruct((BLP, H), f32),
            jax.ShapeDtypeStruct((BLP, H), f32),
            jax.ShapeDtypeStruct((B, H), f32),
        ],
    )(sgf, hid_flat, pos_emb, sadj_p, items_p.reshape(1, BLP),
      items_col, session_len.reshape(B, 1),
      W_ein, b_ein.reshape(1, H), W_eout, b_eout.reshape(1, H),
      w_ih, w_hh, b_ih.reshape(1, 3 * H), b_hh.reshape(1, 3 * H),
      Wq, Wk, Wv, bq, bk, bv,
      out_proj_w, out_proj_b.reshape(1, H), W_target, W3b, W3c)

    scores_full = pl.pallas_call(
        _score_body,
        grid=(MP // MBLK,),
        in_specs=[
            pl.BlockSpec((MBLK, H), lambda i: (i, 0)),
            pl.BlockSpec((BLP, H), lambda i: (0, 0)),
            pl.BlockSpec((BLP, H), lambda i: (0, 0)),
            pl.BlockSpec((B, H), lambda i: (0, 0)),
            pl.BlockSpec((B, LP, 1), lambda i: (0, 0, 0)),
            pl.BlockSpec((H, H), lambda i: (0, 0)),
        ],
        out_specs=pl.BlockSpec((B, MBLK), lambda i: (0, i)),
        out_shape=jax.ShapeDtypeStruct((B, N), f32),
    )(emb, fin, finw, rv, items_col, W3a)

    return scores_full[:, 1:]
